# Initial kernel scaffold; baseline (speedup 1.0000x reference)
import functools

import jax
import jax.numpy as jnp
from jax import lax
from jax.experimental import pallas as pl
from jax.experimental.pallas import tpu as pltpu
from jax.experimental.pallas import tpu_sc as plsc

N = 10000
E = 160000
D = 256
R = 3

NT = 32
RPT = 313
TRASH = RPT
CAP = 6144
CH = 4000
GB = 96
LAST = N - (NT - 1) * RPT
NEG = float("-inf")


def _make_sc_agg(mode):
  mesh = plsc.VectorSubcoreMesh(core_axis_name="c", subcore_axis_name="s")
  init_val = NEG if mode == "max" else 0.0

  @functools.partial(
      pl.kernel,
      out_type=jax.ShapeDtypeStruct((N, D), jnp.float32),
      mesh=mesh,
      scratch_types=[
          pltpu.VMEM((CH,), jnp.int32),
          pltpu.VMEM((CH,), jnp.int32),
          pltpu.VMEM((CAP,), jnp.int32),
          pltpu.VMEM((CAP,), jnp.int32),
          pltpu.VMEM((GB,), jnp.int32),
          pltpu.VMEM((GB,), jnp.int32),
          pltpu.VMEM((GB, D), jnp.float32),
          pltpu.VMEM((RPT + 1, D), jnp.float32),
          pltpu.SemaphoreType.DMA,
      ],
  )
  def sc_agg(xw_hbm, rows_hbm, dst_hbm, out_hbm,
             dstbuf, rowbuf, rows_sel, ld_sel, rowsg, ldg, gbuf, acc, sem):
    wid = lax.axis_index("s") * 2 + lax.axis_index("c")
    lo = wid * RPT
    hi = lo + RPT

    def init_acc(j, carry):
      for k in range(D // 16):
        acc[j, pl.ds(k * 16, 16)] = jnp.full((16,), init_val, jnp.float32)
      return carry
    lax.fori_loop(0, RPT + 1, init_acc, 0)

    def init_sel(j, carry):
      ld_sel[pl.ds(j * 16, 16)] = jnp.full((16,), TRASH, jnp.int32)
      rows_sel[pl.ds(j * 16, 16)] = jnp.zeros((16,), jnp.int32)
      return carry
    lax.fori_loop(0, CAP // 16, init_sel, 0)

    def chunk_body(c, cnt):
      pltpu.sync_copy(dst_hbm.at[pl.ds(c * CH, CH)], dstbuf)
      pltpu.sync_copy(rows_hbm.at[pl.ds(c * CH, CH)], rowbuf)
      def vreg_body(v, cnt):
        sl = pl.ds(v * 16, 16)
        d = dstbuf[sl]
        r = rowbuf[sl]
        m = (d >= lo) & (d < hi)
        mi = m.astype(jnp.int32)
        idx = cnt + plsc.cumsum(mi) - 1
        ok = m & (idx < CAP)
        plsc.store_scatter(ld_sel, [idx], d - lo, mask=ok)
        plsc.store_scatter(rows_sel, [idx], r, mask=ok)
        return cnt + jnp.sum(mi)
      return lax.fori_loop(0, CH // 16, vreg_body, cnt)
    cnt = lax.fori_loop(0, E // CH, chunk_body, jnp.int32(0))

    ngrp = (cnt + (GB - 1)) // GB

    def grp_body(g, carry):
      pltpu.sync_copy(rows_sel.at[pl.ds(g * GB, GB)], rowsg)
      pltpu.sync_copy(ld_sel.at[pl.ds(g * GB, GB)], ldg)
      pltpu.async_copy(xw_hbm.at[rowsg], gbuf, sem).wait()
      if mode == "add":
        pltpu.sync_copy(gbuf, acc.at[ldg], add=True)
      else:
        def sub_body(q, c2):
          ldv = ldg[pl.ds(q * 16, 16)]
          for i in range(16):
            ldi = ldv[i]
            row = q * 16 + i
            for k in range(D // 16):
              sl = pl.ds(k * 16, 16)
              acc[ldi, sl] = jnp.maximum(acc[ldi, sl], gbuf[row, sl])
          return c2
        lax.fori_loop(0, GB // 16, sub_body, 0)
      return carry
    lax.fori_loop(0, ngrp, grp_body, 0)

    @pl.when(wid < NT - 1)
    def _():
      pltpu.sync_copy(acc.at[pl.ds(0, RPT)], out_hbm.at[pl.ds(lo, RPT)])

    @pl.when(wid == NT - 1)
    def _():
      pltpu.sync_copy(acc.at[pl.ds(0, LAST)], out_hbm.at[pl.ds(lo, LAST)])

  return sc_agg


_sc_max = _make_sc_agg("max")
_sc_add = _make_sc_agg("add")

BN = 1000


def _tc_layer1(x, Wrel1, Wroot1, b1):
  def body(x_ref, wr_ref, wro_ref, b_ref, xw_ref, xr_ref):
    xb = x_ref[...]
    for r in range(R):
      xw_ref[r] = jnp.dot(xb, wr_ref[r], preferred_element_type=jnp.float32)
    xr_ref[...] = (jnp.dot(xb, wro_ref[...], preferred_element_type=jnp.float32)
                   + b_ref[...])

  return pl.pallas_call(
      body,
      grid=(N // BN,),
      in_specs=[
          pl.BlockSpec((BN, D), lambda i: (i, 0)),
          pl.BlockSpec((R, D, D), lambda i: (0, 0, 0)),
          pl.BlockSpec((D, D), lambda i: (0, 0)),
          pl.BlockSpec((1, D), lambda i: (0, 0)),
      ],
      out_specs=[
          pl.BlockSpec((R, BN, D), lambda i: (0, i, 0)),
          pl.BlockSpec((BN, D), lambda i: (i, 0)),
      ],
      out_shape=[
          jax.ShapeDtypeStruct((R, N, D), jnp.float32),
          jax.ShapeDtypeStruct((N, D), jnp.float32),
      ],
  )(x, Wrel1, Wroot1, b1.reshape(1, D))


def _layer_norm_in(h, g, b):
  mu = jnp.mean(h, axis=1, keepdims=True)
  var = jnp.mean((h - mu) ** 2, axis=1, keepdims=True)
  return (h - mu) / jnp.sqrt(var + 1e-5) * g + b


def _tc_mid(agg1, xroot1, g1, be1, Wrel2, Wroot2, b2):
  def body(a_ref, xr_ref, g_ref, be_ref, wr_ref, wro_ref, b_ref,
           xw_ref, hr_ref):
    a = a_ref[...]
    a = jnp.where(a == NEG, 0.0, a)
    h = a + xr_ref[...]
    h = _layer_norm_in(h, g_ref[...], be_ref[...])
    h = jnp.maximum(h, 0.0)
    for r in range(R):
      xw_ref[r] = jnp.dot(h, wr_ref[r], preferred_element_type=jnp.float32)
    hr_ref[...] = (jnp.dot(h, wro_ref[...], preferred_element_type=jnp.float32)
                   + b_ref[...])

  return pl.pallas_call(
      body,
      grid=(N // BN,),
      in_specs=[
          pl.BlockSpec((BN, D), lambda i: (i, 0)),
          pl.BlockSpec((BN, D), lambda i: (i, 0)),
          pl.BlockSpec((1, D), lambda i: (0, 0)),
          pl.BlockSpec((1, D), lambda i: (0, 0)),
          pl.BlockSpec((R, D, D), lambda i: (0, 0, 0)),
          pl.BlockSpec((D, D), lambda i: (0, 0)),
          pl.BlockSpec((1, D), lambda i: (0, 0)),
      ],
      out_specs=[
          pl.BlockSpec((R, BN, D), lambda i: (0, i, 0)),
          pl.BlockSpec((BN, D), lambda i: (i, 0)),
      ],
      out_shape=[
          jax.ShapeDtypeStruct((R, N, D), jnp.float32),
          jax.ShapeDtypeStruct((N, D), jnp.float32),
      ],
  )(agg1, xroot1, g1.reshape(1, D), be1.reshape(1, D), Wrel2, Wroot2,
    b2.reshape(1, D))


def _tc_out(agg2, hroot2, g2, be2):
  def body(a_ref, hr_ref, g_ref, be_ref, o_ref):
    z = _layer_norm_in(a_ref[...] + hr_ref[...], g_ref[...], be_ref[...])
    z = z - jnp.max(z, axis=1, keepdims=True)
    o_ref[...] = z - jnp.log(jnp.sum(jnp.exp(z), axis=1, keepdims=True))

  return pl.pallas_call(
      body,
      grid=(N // BN,),
      in_specs=[
          pl.BlockSpec((BN, D), lambda i: (i, 0)),
          pl.BlockSpec((BN, D), lambda i: (i, 0)),
          pl.BlockSpec((1, D), lambda i: (0, 0)),
          pl.BlockSpec((1, D), lambda i: (0, 0)),
      ],
      out_specs=pl.BlockSpec((BN, D), lambda i: (i, 0)),
      out_shape=jax.ShapeDtypeStruct((N, D), jnp.float32),
  )(agg2, hroot2, g2.reshape(1, D), be2.reshape(1, D))


@jax.jit
def kernel(x, edge_index, edge_type, Wrel1, Wroot1, b1, g1, be1,
           Wrel2, Wroot2, b2, g2, be2):
  src = edge_index[0]
  dst = edge_index[1]
  rows = edge_type * N + src

  xw1, xroot1 = _tc_layer1(x, Wrel1, Wroot1, b1)
  agg1 = _sc_max(xw1.reshape(R * N, D), rows, dst)
  xw2, hroot2 = _tc_mid(agg1, xroot1, g1, be1, Wrel2, Wroot2, b2)
  agg2 = _sc_add(xw2.reshape(R * N, D), rows, dst)
  return _tc_out(agg2, hroot2, g2, be2)


# speedup vs baseline: 3.2574x; 3.2574x over previous
import functools

import jax
import jax.numpy as jnp
from jax import lax
from jax.experimental import pallas as pl
from jax.experimental.pallas import tpu as pltpu
from jax.experimental.pallas import tpu_sc as plsc

N = 10000
E = 160000
D = 256
R = 3

NT = 32
RPT = 320
TRASH = RPT
CAP = 6400
CH = 3200
GB = 96
LAST = N - (NT - 1) * RPT
NEG = float("-inf")


def _make_sc_agg(mode):
  mesh = plsc.VectorSubcoreMesh(core_axis_name="c", subcore_axis_name="s")
  init_val = NEG if mode == "max" else 0.0

  combine = jnp.maximum if mode == "max" else jnp.add

  @functools.partial(
      pl.kernel,
      out_type=jax.ShapeDtypeStruct((N, D), jnp.float32),
      mesh=mesh,
      scratch_types=[
          pltpu.VMEM((CH,), jnp.int32),
          pltpu.VMEM((CH,), jnp.int32),
          pltpu.VMEM((CAP,), jnp.int32),
          pltpu.VMEM((CAP,), jnp.int32),
          pltpu.VMEM((GB, D), jnp.float32),
          pltpu.VMEM((RPT + 1, D), jnp.float32),
          pltpu.SemaphoreType.DMA,
      ],
      compiler_params=pltpu.CompilerParams(needs_layout_passes=False),
  )
  def sc_agg(xw_hbm, rows_hbm, dst_hbm, out_hbm,
             dstbuf, rowbuf, rows_sel, ld_sel, gbuf, acc, sem):
    wid = lax.axis_index("s") * 2 + lax.axis_index("c")
    lo = wid * RPT
    hi = lo + RPT

    def init_acc(j, carry):
      for k in range(D // 16):
        acc[j, pl.ds(k * 16, 16)] = jnp.full((16,), init_val, jnp.float32)
      return carry
    lax.fori_loop(0, RPT + 1, init_acc, 0)

    def init_sel(j, carry):
      ld_sel[pl.ds(j * 16, 16)] = jnp.full((16,), TRASH, jnp.int32)
      rows_sel[pl.ds(j * 16, 16)] = jnp.zeros((16,), jnp.int32)
      return carry
    lax.fori_loop(0, CAP // 16, init_sel, 0)

    def chunk_body(c, cnt):
      pltpu.sync_copy(dst_hbm.at[pl.ds(c * CH, CH)], dstbuf)
      pltpu.sync_copy(rows_hbm.at[pl.ds(c * CH, CH)], rowbuf)
      def vreg_body(v, cnt):
        sl = pl.ds(v * 16, 16)
        d = dstbuf[sl]
        r = rowbuf[sl]
        m = (d >= lo) & (d < hi)
        mi = m.astype(jnp.int32)
        idx = cnt + plsc.cumsum(mi) - 1
        ok = m & (idx < CAP)
        plsc.store_scatter(ld_sel, [idx], d - lo, mask=ok)
        plsc.store_scatter(rows_sel, [idx], r, mask=ok)
        return cnt + jnp.sum(mi)
      return lax.fori_loop(0, CH // 16, vreg_body, cnt)
    cnt = lax.fori_loop(0, E // CH, chunk_body, jnp.int32(0))

    ngrp = (cnt + (GB - 1)) // GB

    def grp_body(g, carry):
      pltpu.async_copy(xw_hbm.at[rows_sel.at[pl.ds(g * GB, GB)]],
                       gbuf, sem).wait()
      def sub_body(q, c2):
        ldv = ld_sel[pl.ds(g * GB + q * 16, 16)]
        for i in range(16):
          ldi = ldv[i]
          row = q * 16 + i
          for k in range(D // 16):
            sl = pl.ds(k * 16, 16)
            acc[ldi, sl] = combine(acc[ldi, sl], gbuf[row, sl])
        return c2
      lax.fori_loop(0, GB // 16, sub_body, 0)
      return carry
    lax.fori_loop(0, ngrp, grp_body, 0)

    @pl.when(wid < NT - 1)
    def _():
      pltpu.sync_copy(acc.at[pl.ds(0, RPT)], out_hbm.at[pl.ds(lo, RPT)])

    @pl.when(wid == NT - 1)
    def _():
      pltpu.sync_copy(acc.at[pl.ds(0, LAST)], out_hbm.at[pl.ds(lo, LAST)])

  return sc_agg


_sc_max = _make_sc_agg("max")
_sc_add = _make_sc_agg("add")

BN = 1000


def _tc_layer1(x, Wrel1, Wroot1, b1):
  def body(x_ref, wr_ref, wro_ref, b_ref, xw_ref, xr_ref):
    xb = x_ref[...]
    for r in range(R):
      xw_ref[r] = jnp.dot(xb, wr_ref[r], preferred_element_type=jnp.float32)
    xr_ref[...] = (jnp.dot(xb, wro_ref[...], preferred_element_type=jnp.float32)
                   + b_ref[...])

  return pl.pallas_call(
      body,
      grid=(N // BN,),
      in_specs=[
          pl.BlockSpec((BN, D), lambda i: (i, 0)),
          pl.BlockSpec((R, D, D), lambda i: (0, 0, 0)),
          pl.BlockSpec((D, D), lambda i: (0, 0)),
          pl.BlockSpec((1, D), lambda i: (0, 0)),
      ],
      out_specs=[
          pl.BlockSpec((R, BN, D), lambda i: (0, i, 0)),
          pl.BlockSpec((BN, D), lambda i: (i, 0)),
      ],
      out_shape=[
          jax.ShapeDtypeStruct((R, N, D), jnp.float32),
          jax.ShapeDtypeStruct((N, D), jnp.float32),
      ],
  )(x, Wrel1, Wroot1, b1.reshape(1, D))


def _layer_norm_in(h, g, b):
  mu = jnp.mean(h, axis=1, keepdims=True)
  var = jnp.mean((h - mu) ** 2, axis=1, keepdims=True)
  return (h - mu) / jnp.sqrt(var + 1e-5) * g + b


def _tc_mid(agg1, xroot1, g1, be1, Wrel2, Wroot2, b2):
  def body(a_ref, xr_ref, g_ref, be_ref, wr_ref, wro_ref, b_ref,
           xw_ref, hr_ref):
    a = a_ref[...]
    a = jnp.where(a == NEG, 0.0, a)
    h = a + xr_ref[...]
    h = _layer_norm_in(h, g_ref[...], be_ref[...])
    h = jnp.maximum(h, 0.0)
    for r in range(R):
      xw_ref[r] = jnp.dot(h, wr_ref[r], preferred_element_type=jnp.float32)
    hr_ref[...] = (jnp.dot(h, wro_ref[...], preferred_element_type=jnp.float32)
                   + b_ref[...])

  return pl.pallas_call(
      body,
      grid=(N // BN,),
      in_specs=[
          pl.BlockSpec((BN, D), lambda i: (i, 0)),
          pl.BlockSpec((BN, D), lambda i: (i, 0)),
          pl.BlockSpec((1, D), lambda i: (0, 0)),
          pl.BlockSpec((1, D), lambda i: (0, 0)),
          pl.BlockSpec((R, D, D), lambda i: (0, 0, 0)),
          pl.BlockSpec((D, D), lambda i: (0, 0)),
          pl.BlockSpec((1, D), lambda i: (0, 0)),
      ],
      out_specs=[
          pl.BlockSpec((R, BN, D), lambda i: (0, i, 0)),
          pl.BlockSpec((BN, D), lambda i: (i, 0)),
      ],
      out_shape=[
          jax.ShapeDtypeStruct((R, N, D), jnp.float32),
          jax.ShapeDtypeStruct((N, D), jnp.float32),
      ],
  )(agg1, xroot1, g1.reshape(1, D), be1.reshape(1, D), Wrel2, Wroot2,
    b2.reshape(1, D))


def _tc_out(agg2, hroot2, g2, be2):
  def body(a_ref, hr_ref, g_ref, be_ref, o_ref):
    z = _layer_norm_in(a_ref[...] + hr_ref[...], g_ref[...], be_ref[...])
    z = z - jnp.max(z, axis=1, keepdims=True)
    o_ref[...] = z - jnp.log(jnp.sum(jnp.exp(z), axis=1, keepdims=True))

  return pl.pallas_call(
      body,
      grid=(N // BN,),
      in_specs=[
          pl.BlockSpec((BN, D), lambda i: (i, 0)),
          pl.BlockSpec((BN, D), lambda i: (i, 0)),
          pl.BlockSpec((1, D), lambda i: (0, 0)),
          pl.BlockSpec((1, D), lambda i: (0, 0)),
      ],
      out_specs=pl.BlockSpec((BN, D), lambda i: (i, 0)),
      out_shape=jax.ShapeDtypeStruct((N, D), jnp.float32),
  )(agg2, hroot2, g2.reshape(1, D), be2.reshape(1, D))


@jax.jit
def kernel(x, edge_index, edge_type, Wrel1, Wroot1, b1, g1, be1,
           Wrel2, Wroot2, b2, g2, be2):
  src = edge_index[0]
  dst = edge_index[1]
  rows = edge_type * N + src

  xw1, xroot1 = _tc_layer1(x, Wrel1, Wroot1, b1)
  agg1 = _sc_max(xw1.reshape(R * N, D), rows, dst)
  xw2, hroot2 = _tc_mid(agg1, xroot1, g1, be1, Wrel2, Wroot2, b2)
  agg2 = _sc_add(xw2.reshape(R * N, D), rows, dst)
  return _tc_out(agg2, hroot2, g2, be2)
